# baseline (speedup 1.0000x reference)
import functools

import jax
import jax.numpy as jnp
from jax import lax
from jax.experimental import pallas as pl
from jax.experimental.pallas import tpu as pltpu
from jax.experimental.pallas import tpu_sc as plsc


_HOT = 512


def _sc_gather(W, b, sample_ids, targets):
    ns = sample_ids.shape[0]
    bsz = targets.shape[0]
    hid = W.shape[1]
    info = plsc.get_sparse_core_info()
    nw = info.num_cores * info.num_subcores
    per_s = ns // nw
    per_t = bsz // nw
    assert per_s * nw == ns and per_t * nw == bsz
    assert per_s % 16 == 0 and per_t % 8 == 0
    mesh = plsc.VectorSubcoreMesh(core_axis_name="c", subcore_axis_name="s")

    @functools.partial(
        pl.kernel,
        out_type=(
            jax.ShapeDtypeStruct((ns + 8 * nw, hid), jnp.float32),
            jax.ShapeDtypeStruct((bsz, hid), jnp.float32),
            jax.ShapeDtypeStruct((ns + bsz,), jnp.float32),
        ),
        mesh=mesh,
        scratch_types=[
            pltpu.VMEM((per_s,), jnp.int32),
            pltpu.VMEM((per_s,), jnp.int32),
            pltpu.VMEM((per_s,), jnp.int32),
            pltpu.VMEM((per_s,), jnp.int32),
            pltpu.VMEM((per_t,), jnp.int32),
            pltpu.VMEM((per_s, hid), jnp.float32),
            pltpu.VMEM((per_s, hid), jnp.float32),
            pltpu.VMEM((per_t, hid), jnp.float32),
            pltpu.VMEM((per_s,), jnp.float32),
            pltpu.VMEM((per_t,), jnp.float32),
            pltpu.VMEM((per_s,), jnp.float32),
            pltpu.VMEM_SHARED((_HOT, hid), jnp.float32),
            pltpu.VMEM_SHARED((_HOT,), jnp.float32),
            pltpu.SemaphoreType.DMA,
            pltpu.SemaphoreType.DMA,
            pltpu.SemaphoreType.DMA,
            pltpu.SemaphoreType.DMA,
            pltpu.SemaphoreType.DMA,
            pltpu.SemaphoreType.DMA,
        ],
    )
    def gather_kernel(w_hbm, b_hbm, sid_hbm, tgt_hbm,
                      swp_out, trows_out, bias_out,
                      idx_s, idx_hot, idx_cold, pos_hot, idx_t,
                      buf_c, buf_h, buf_t, bias_s, bias_t, bias_h,
                      hot_rows, b_hot_sh,
                      sem_c, sem_h, sem_t, sem_bs, sem_bt, sem_bh):
        sid = lax.axis_index("s")
        wid = sid * info.num_cores + lax.axis_index("c")
        base_s = wid * per_s
        base_t = wid * per_t

        @pl.when(sid == 0)
        def _stage():
            pltpu.sync_copy(w_hbm.at[pl.ds(0, _HOT)], hot_rows)
            pltpu.sync_copy(b_hbm.at[pl.ds(0, _HOT)], b_hot_sh)

        pltpu.sync_copy(sid_hbm.at[pl.ds(base_s, per_s)], idx_s)
        pltpu.sync_copy(tgt_hbm.at[pl.ds(base_t, per_t)], idx_t)
        dump = ns + wid * 8
        for c in range(per_s // 16):
            ids16 = idx_s[pl.ds(c * 16, 16)]
            sub16 = base_s + c * 16 + lax.broadcasted_iota(
                jnp.int32, (16,), 0)
            hot = ids16 < _HOT
            idx_hot[pl.ds(c * 16, 16)] = jnp.where(hot, ids16, 0)
            idx_cold[pl.ds(c * 16, 16)] = jnp.where(hot, sub16, ids16)
            pos_hot[pl.ds(c * 16, 16)] = jnp.where(hot, sub16, dump)
        plsc.subcore_barrier()
        cp_c = pltpu.async_copy(w_hbm.at[idx_cold], buf_c, sem_c)
        cp_h = pltpu.async_copy(hot_rows.at[idx_hot], buf_h, sem_h)
        cp_t = pltpu.async_copy(w_hbm.at[idx_t], buf_t, sem_t)
        cp_bs = pltpu.async_copy(b_hbm.at[idx_cold], bias_s, sem_bs)
        cp_bh = pltpu.async_copy(b_hot_sh.at[idx_hot], bias_h, sem_bh)
        cp_bt = pltpu.async_copy(b_hbm.at[idx_t], bias_t, sem_bt)
        cp_c.wait()
        pltpu.sync_copy(buf_c, swp_out.at[pl.ds(base_s, per_s)])
        cp_h.wait()
        cp_w = pltpu.async_copy(buf_h, swp_out.at[pos_hot], sem_c)
        cp_t.wait()
        pltpu.sync_copy(buf_t, trows_out.at[pl.ds(base_t, per_t)])
        cp_bs.wait()
        cp_bh.wait()
        for c in range(per_s // 16):
            ids16 = idx_s[pl.ds(c * 16, 16)]
            hot = ids16 < _HOT
            cold16 = bias_s[pl.ds(c * 16, 16)]
            hotv = bias_h[pl.ds(c * 16, 16)]
            bias_s[pl.ds(c * 16, 16)] = jnp.where(hot, hotv, cold16)
        pltpu.sync_copy(bias_s, bias_out.at[pl.ds(base_s, per_s)])
        cp_bt.wait()
        pltpu.sync_copy(bias_t, bias_out.at[pl.ds(ns + base_t, per_t)])
        cp_w.wait()

    return gather_kernel(W, b, sample_ids, targets)


def _tc_logits_t(output, targets2, swp, trows, class_vecs, true_b2, true_f2,
                 bt):
    b, hid = output.shape
    ns = class_vecs.shape[0] - 1

    def body(x_ref, tgt_ref, sw_ref, tw_ref, cv_ref, tb_ref, tf_ref, o_ref):
        x = x_ref[...]
        sw = jnp.concatenate(
            [jnp.zeros((1, hid), jnp.float32), sw_ref[...]], axis=0)
        res = lax.dot_general(
            sw.astype(jnp.bfloat16), x.astype(jnp.bfloat16),
            (((1,), (1,)), ((), ())),
            preferred_element_type=jnp.float32)
        sid = cv_ref[:, 0:1]
        sb = lax.bitcast_convert_type(cv_ref[:, 1:2], jnp.float32)
        sf = lax.bitcast_convert_type(cv_ref[:, 2:3], jnp.float32)
        res = res + (sb - jnp.log(sf))
        acc = sid == tgt_ref[...]
        res = jnp.where(acc, jnp.float32(-1e37), res)
        ones = jnp.ones((1, hid), dtype=jnp.float32)
        tl = lax.dot_general(
            ones, x * tw_ref[...], (((1,), (1,)), ((), ())),
            preferred_element_type=jnp.float32)
        tl = tl + tb_ref[...] - jnp.log(tf_ref[...])
        o_ref[0:1, :] = tl
        o_ref[1:, :] = res[1:, :]

    grid = (b // bt,)
    return pl.pallas_call(
        body,
        grid=grid,
        in_specs=[
            pl.BlockSpec((bt, hid), lambda j: (j, 0)),
            pl.BlockSpec((1, bt), lambda j: (0, j)),
            pl.BlockSpec((ns, hid), lambda j: (0, 0)),
            pl.BlockSpec((bt, hid), lambda j: (j, 0)),
            pl.BlockSpec((1 + ns, 3), lambda j: (0, 0)),
            pl.BlockSpec((1, bt), lambda j: (0, j)),
            pl.BlockSpec((1, bt), lambda j: (0, j)),
        ],
        out_specs=pl.BlockSpec((1 + ns, bt), lambda j: (0, j)),
        out_shape=jax.ShapeDtypeStruct((1 + ns, b), jnp.float32),
    )(output, targets2, swp, trows, class_vecs, true_b2, true_f2)


def kernel(output, targets, W, b, sample_ids, true_freq, sample_freq):
    bsz, hid = output.shape
    ns = sample_ids.shape[0]
    swp, trows, bias = _sc_gather(W, b, sample_ids, targets)
    neg1 = jnp.full((1,), -1, dtype=jnp.int32)
    zero1 = jnp.zeros((1,), dtype=jnp.int32)
    one1 = lax.bitcast_convert_type(
        jnp.full((1,), 1.0, jnp.float32), jnp.int32)
    class_vecs = jnp.stack(
        [jnp.concatenate([neg1, sample_ids]),
         jnp.concatenate([zero1,
                          lax.bitcast_convert_type(bias[:ns], jnp.int32)]),
         jnp.concatenate([one1,
                          lax.bitcast_convert_type(sample_freq, jnp.int32)])],
        axis=1)
    logits_t = _tc_logits_t(
        output,
        targets.reshape(1, bsz),
        swp,
        trows,
        class_vecs,
        bias[ns:].reshape(1, bsz),
        true_freq.reshape(1, bsz),
        bt=512,
    )
    logits = logits_t.T
    new_targets = jnp.zeros((bsz,), dtype=jnp.int32)
    return logits, new_targets


# speedup vs baseline: 1.0084x; 1.0084x over previous
import functools

import jax
import jax.numpy as jnp
from jax import lax
from jax.experimental import pallas as pl
from jax.experimental.pallas import tpu as pltpu
from jax.experimental.pallas import tpu_sc as plsc


_HOT = 2048


def _sc_gather(W, b, sample_ids, targets):
    ns = sample_ids.shape[0]
    bsz = targets.shape[0]
    hid = W.shape[1]
    info = plsc.get_sparse_core_info()
    nw = info.num_cores * info.num_subcores
    per_s = ns // nw
    per_t = bsz // nw
    assert per_s * nw == ns and per_t * nw == bsz
    assert per_s % 16 == 0 and per_t % 8 == 0
    mesh = plsc.VectorSubcoreMesh(core_axis_name="c", subcore_axis_name="s")

    @functools.partial(
        pl.kernel,
        out_type=(
            jax.ShapeDtypeStruct((ns + 8 * nw, hid), jnp.float32),
            jax.ShapeDtypeStruct((bsz, hid), jnp.float32),
            jax.ShapeDtypeStruct((ns + bsz,), jnp.float32),
        ),
        mesh=mesh,
        scratch_types=[
            pltpu.VMEM((per_s,), jnp.int32),
            pltpu.VMEM((per_s,), jnp.int32),
            pltpu.VMEM((per_s,), jnp.int32),
            pltpu.VMEM((per_s,), jnp.int32),
            pltpu.VMEM((per_t,), jnp.int32),
            pltpu.VMEM((per_s, hid), jnp.float32),
            pltpu.VMEM((per_s, hid), jnp.float32),
            pltpu.VMEM((per_t, hid), jnp.float32),
            pltpu.VMEM((per_s,), jnp.float32),
            pltpu.VMEM((per_t,), jnp.float32),
            pltpu.VMEM((per_s,), jnp.float32),
            pltpu.VMEM_SHARED((_HOT, hid), jnp.float32),
            pltpu.VMEM_SHARED((_HOT,), jnp.float32),
            pltpu.SemaphoreType.DMA,
            pltpu.SemaphoreType.DMA,
            pltpu.SemaphoreType.DMA,
            pltpu.SemaphoreType.DMA,
            pltpu.SemaphoreType.DMA,
            pltpu.SemaphoreType.DMA,
        ],
    )
    def gather_kernel(w_hbm, b_hbm, sid_hbm, tgt_hbm,
                      swp_out, trows_out, bias_out,
                      idx_s, idx_hot, idx_cold, pos_hot, idx_t,
                      buf_c, buf_h, buf_t, bias_s, bias_t, bias_h,
                      hot_rows, b_hot_sh,
                      sem_c, sem_h, sem_t, sem_bs, sem_bt, sem_bh):
        sid = lax.axis_index("s")
        wid = sid * info.num_cores + lax.axis_index("c")
        base_s = wid * per_s
        base_t = wid * per_t

        @pl.when(sid == 0)
        def _stage():
            pltpu.sync_copy(w_hbm.at[pl.ds(0, _HOT)], hot_rows)
            pltpu.sync_copy(b_hbm.at[pl.ds(0, _HOT)], b_hot_sh)

        pltpu.sync_copy(sid_hbm.at[pl.ds(base_s, per_s)], idx_s)
        pltpu.sync_copy(tgt_hbm.at[pl.ds(base_t, per_t)], idx_t)
        dump = ns + wid * 8
        for c in range(per_s // 16):
            ids16 = idx_s[pl.ds(c * 16, 16)]
            sub16 = base_s + c * 16 + lax.broadcasted_iota(
                jnp.int32, (16,), 0)
            hot = ids16 < _HOT
            idx_hot[pl.ds(c * 16, 16)] = jnp.where(hot, ids16, 0)
            idx_cold[pl.ds(c * 16, 16)] = jnp.where(hot, sub16, ids16)
            pos_hot[pl.ds(c * 16, 16)] = jnp.where(hot, sub16, dump)
        plsc.subcore_barrier()
        cp_c = pltpu.async_copy(w_hbm.at[idx_cold], buf_c, sem_c)
        cp_h = pltpu.async_copy(hot_rows.at[idx_hot], buf_h, sem_h)
        cp_t = pltpu.async_copy(w_hbm.at[idx_t], buf_t, sem_t)
        cp_bs = pltpu.async_copy(b_hbm.at[idx_cold], bias_s, sem_bs)
        cp_bh = pltpu.async_copy(b_hot_sh.at[idx_hot], bias_h, sem_bh)
        cp_bt = pltpu.async_copy(b_hbm.at[idx_t], bias_t, sem_bt)
        cp_c.wait()
        pltpu.sync_copy(buf_c, swp_out.at[pl.ds(base_s, per_s)])
        cp_h.wait()
        cp_w = pltpu.async_copy(buf_h, swp_out.at[pos_hot], sem_c)
        cp_t.wait()
        pltpu.sync_copy(buf_t, trows_out.at[pl.ds(base_t, per_t)])
        cp_bs.wait()
        cp_bh.wait()
        for c in range(per_s // 16):
            ids16 = idx_s[pl.ds(c * 16, 16)]
            hot = ids16 < _HOT
            cold16 = bias_s[pl.ds(c * 16, 16)]
            hotv = bias_h[pl.ds(c * 16, 16)]
            bias_s[pl.ds(c * 16, 16)] = jnp.where(hot, hotv, cold16)
        pltpu.sync_copy(bias_s, bias_out.at[pl.ds(base_s, per_s)])
        cp_bt.wait()
        pltpu.sync_copy(bias_t, bias_out.at[pl.ds(ns + base_t, per_t)])
        cp_w.wait()

    return gather_kernel(W, b, sample_ids, targets)


def _tc_logits_t(output, targets2, swp, trows, class_vecs, true_b2, true_f2,
                 bt):
    b, hid = output.shape
    ns = class_vecs.shape[0] - 1

    def body(x_ref, tgt_ref, sw_ref, tw_ref, cv_ref, tb_ref, tf_ref, o_ref):
        x = x_ref[...]
        sw = jnp.concatenate(
            [jnp.zeros((1, hid), jnp.float32), sw_ref[...]], axis=0)
        res = lax.dot_general(
            sw.astype(jnp.bfloat16), x.astype(jnp.bfloat16),
            (((1,), (1,)), ((), ())),
            preferred_element_type=jnp.float32)
        sid = cv_ref[:, 0:1]
        sb = lax.bitcast_convert_type(cv_ref[:, 1:2], jnp.float32)
        sf = lax.bitcast_convert_type(cv_ref[:, 2:3], jnp.float32)
        res = res + (sb - jnp.log(sf))
        acc = sid == tgt_ref[...]
        res = jnp.where(acc, jnp.float32(-1e37), res)
        ones = jnp.ones((1, hid), dtype=jnp.float32)
        tl = lax.dot_general(
            ones, x * tw_ref[...], (((1,), (1,)), ((), ())),
            preferred_element_type=jnp.float32)
        tl = tl + tb_ref[...] - jnp.log(tf_ref[...])
        o_ref[0:1, :] = tl
        o_ref[1:, :] = res[1:, :]

    grid = (b // bt,)
    return pl.pallas_call(
        body,
        grid=grid,
        in_specs=[
            pl.BlockSpec((bt, hid), lambda j: (j, 0)),
            pl.BlockSpec((1, bt), lambda j: (0, j)),
            pl.BlockSpec((ns, hid), lambda j: (0, 0)),
            pl.BlockSpec((bt, hid), lambda j: (j, 0)),
            pl.BlockSpec((1 + ns, 3), lambda j: (0, 0)),
            pl.BlockSpec((1, bt), lambda j: (0, j)),
            pl.BlockSpec((1, bt), lambda j: (0, j)),
        ],
        out_specs=pl.BlockSpec((1 + ns, bt), lambda j: (0, j)),
        out_shape=jax.ShapeDtypeStruct((1 + ns, b), jnp.float32),
    )(output, targets2, swp, trows, class_vecs, true_b2, true_f2)


def kernel(output, targets, W, b, sample_ids, true_freq, sample_freq):
    bsz, hid = output.shape
    ns = sample_ids.shape[0]
    swp, trows, bias = _sc_gather(W, b, sample_ids, targets)
    neg1 = jnp.full((1,), -1, dtype=jnp.int32)
    zero1 = jnp.zeros((1,), dtype=jnp.int32)
    one1 = lax.bitcast_convert_type(
        jnp.full((1,), 1.0, jnp.float32), jnp.int32)
    class_vecs = jnp.stack(
        [jnp.concatenate([neg1, sample_ids]),
         jnp.concatenate([zero1,
                          lax.bitcast_convert_type(bias[:ns], jnp.int32)]),
         jnp.concatenate([one1,
                          lax.bitcast_convert_type(sample_freq, jnp.int32)])],
        axis=1)
    logits_t = _tc_logits_t(
        output,
        targets.reshape(1, bsz),
        swp,
        trows,
        class_vecs,
        bias[ns:].reshape(1, bsz),
        true_freq.reshape(1, bsz),
        bt=512,
    )
    logits = logits_t.T
    new_targets = jnp.zeros((bsz,), dtype=jnp.int32)
    return logits, new_targets


# speedup vs baseline: 1.0394x; 1.0307x over previous
import functools

import jax
import jax.numpy as jnp
from jax import lax
from jax.experimental import pallas as pl
from jax.experimental.pallas import tpu as pltpu
from jax.experimental.pallas import tpu_sc as plsc


_HOT = 2048


def _sc_gather(W, b, sample_ids, targets):
    ns = sample_ids.shape[0]
    bsz = targets.shape[0]
    hid = W.shape[1]
    info = plsc.get_sparse_core_info()
    nw = info.num_cores * info.num_subcores
    per_s = ns // nw
    per_t = bsz // nw
    assert per_s * nw == ns and per_t * nw == bsz
    assert per_s % 16 == 0 and per_t % 8 == 0
    mesh = plsc.VectorSubcoreMesh(core_axis_name="c", subcore_axis_name="s")

    @functools.partial(
        pl.kernel,
        out_type=(
            jax.ShapeDtypeStruct((ns + 8 * nw, hid), jnp.float32),
            jax.ShapeDtypeStruct((bsz, hid), jnp.float32),
            jax.ShapeDtypeStruct((ns + bsz,), jnp.float32),
        ),
        mesh=mesh,
        scratch_types=[
            pltpu.VMEM((per_s,), jnp.int32),
            pltpu.VMEM((per_s,), jnp.int32),
            pltpu.VMEM((per_s,), jnp.int32),
            pltpu.VMEM((per_s,), jnp.int32),
            pltpu.VMEM((per_t,), jnp.int32),
            pltpu.VMEM((per_s, hid), jnp.float32),
            pltpu.VMEM((per_s, hid), jnp.float32),
            pltpu.VMEM((per_t, hid), jnp.float32),
            pltpu.VMEM((per_s,), jnp.float32),
            pltpu.VMEM((per_t,), jnp.float32),
            pltpu.VMEM((per_s,), jnp.float32),
            pltpu.VMEM_SHARED((_HOT, hid), jnp.float32),
            pltpu.VMEM_SHARED((_HOT,), jnp.float32),
            pltpu.SemaphoreType.DMA,
            pltpu.SemaphoreType.DMA,
            pltpu.SemaphoreType.DMA,
            pltpu.SemaphoreType.DMA,
            pltpu.SemaphoreType.DMA,
            pltpu.SemaphoreType.DMA,
        ],
    )
    def gather_kernel(w_hbm, b_hbm, sid_hbm, tgt_hbm,
                      swp_out, trows_out, bias_out,
                      idx_s, idx_hot, idx_cold, pos_hot, idx_t,
                      buf_c, buf_h, buf_t, bias_s, bias_t, bias_h,
                      hot_rows, b_hot_sh,
                      sem_c, sem_h, sem_t, sem_bs, sem_bt, sem_bh):
        sid = lax.axis_index("s")
        wid = sid * info.num_cores + lax.axis_index("c")
        base_s = wid * per_s
        base_t = wid * per_t

        @pl.when(sid == 0)
        def _stage():
            pltpu.sync_copy(w_hbm.at[pl.ds(0, _HOT)], hot_rows)
            pltpu.sync_copy(b_hbm.at[pl.ds(0, _HOT)], b_hot_sh)

        pltpu.sync_copy(sid_hbm.at[pl.ds(base_s, per_s)], idx_s)
        pltpu.sync_copy(tgt_hbm.at[pl.ds(base_t, per_t)], idx_t)
        dump = ns + wid * 8
        for c in range(per_s // 16):
            ids16 = idx_s[pl.ds(c * 16, 16)]
            sub16 = base_s + c * 16 + lax.broadcasted_iota(
                jnp.int32, (16,), 0)
            hot = ids16 < _HOT
            idx_hot[pl.ds(c * 16, 16)] = jnp.where(hot, ids16, 0)
            idx_cold[pl.ds(c * 16, 16)] = jnp.where(hot, sub16, ids16)
            pos_hot[pl.ds(c * 16, 16)] = jnp.where(
                hot, sub16, dump + (sub16 & 7))
        plsc.subcore_barrier()
        cp_c = pltpu.async_copy(w_hbm.at[idx_cold], buf_c, sem_c)
        cp_h = pltpu.async_copy(hot_rows.at[idx_hot], buf_h, sem_h)
        cp_t = pltpu.async_copy(w_hbm.at[idx_t], buf_t, sem_t)
        cp_bs = pltpu.async_copy(b_hbm.at[idx_cold], bias_s, sem_bs)
        cp_bh = pltpu.async_copy(b_hot_sh.at[idx_hot], bias_h, sem_bh)
        cp_bt = pltpu.async_copy(b_hbm.at[idx_t], bias_t, sem_bt)
        cp_c.wait()
        pltpu.sync_copy(buf_c, swp_out.at[pl.ds(base_s, per_s)])
        cp_h.wait()
        cp_w = pltpu.async_copy(buf_h, swp_out.at[pos_hot], sem_c)
        cp_t.wait()
        pltpu.sync_copy(buf_t, trows_out.at[pl.ds(base_t, per_t)])
        cp_bs.wait()
        cp_bh.wait()
        for c in range(per_s // 16):
            ids16 = idx_s[pl.ds(c * 16, 16)]
            hot = ids16 < _HOT
            cold16 = bias_s[pl.ds(c * 16, 16)]
            hotv = bias_h[pl.ds(c * 16, 16)]
            bias_s[pl.ds(c * 16, 16)] = jnp.where(hot, hotv, cold16)
        pltpu.sync_copy(bias_s, bias_out.at[pl.ds(base_s, per_s)])
        cp_bt.wait()
        pltpu.sync_copy(bias_t, bias_out.at[pl.ds(ns + base_t, per_t)])
        cp_w.wait()

    return gather_kernel(W, b, sample_ids, targets)


def _tc_logits_t(output, targets2, swp, trows, class_vecs, true_b2, true_f2,
                 bt):
    b, hid = output.shape
    ns = class_vecs.shape[0] - 1

    def body(x_ref, tgt_ref, sw_ref, tw_ref, cv_ref, tb_ref, tf_ref, o_ref):
        x = x_ref[...]
        sw = jnp.concatenate(
            [jnp.zeros((1, hid), jnp.float32), sw_ref[...]], axis=0)
        res = lax.dot_general(
            sw.astype(jnp.bfloat16), x.astype(jnp.bfloat16),
            (((1,), (1,)), ((), ())),
            preferred_element_type=jnp.float32)
        sid = cv_ref[:, 0:1]
        sb = lax.bitcast_convert_type(cv_ref[:, 1:2], jnp.float32)
        sf = lax.bitcast_convert_type(cv_ref[:, 2:3], jnp.float32)
        res = res + (sb - jnp.log(sf))
        acc = sid == tgt_ref[...]
        res = jnp.where(acc, jnp.float32(-1e37), res)
        ones = jnp.ones((1, hid), dtype=jnp.float32)
        tl = lax.dot_general(
            ones, x * tw_ref[...], (((1,), (1,)), ((), ())),
            preferred_element_type=jnp.float32)
        tl = tl + tb_ref[...] - jnp.log(tf_ref[...])
        o_ref[0:1, :] = tl
        o_ref[1:, :] = res[1:, :]

    grid = (b // bt,)
    return pl.pallas_call(
        body,
        grid=grid,
        in_specs=[
            pl.BlockSpec((bt, hid), lambda j: (j, 0)),
            pl.BlockSpec((1, bt), lambda j: (0, j)),
            pl.BlockSpec((ns, hid), lambda j: (0, 0)),
            pl.BlockSpec((bt, hid), lambda j: (j, 0)),
            pl.BlockSpec((1 + ns, 3), lambda j: (0, 0)),
            pl.BlockSpec((1, bt), lambda j: (0, j)),
            pl.BlockSpec((1, bt), lambda j: (0, j)),
        ],
        out_specs=pl.BlockSpec((1 + ns, bt), lambda j: (0, j)),
        out_shape=jax.ShapeDtypeStruct((1 + ns, b), jnp.float32),
    )(output, targets2, swp, trows, class_vecs, true_b2, true_f2)


def kernel(output, targets, W, b, sample_ids, true_freq, sample_freq):
    bsz, hid = output.shape
    ns = sample_ids.shape[0]
    swp, trows, bias = _sc_gather(W, b, sample_ids, targets)
    neg1 = jnp.full((1,), -1, dtype=jnp.int32)
    zero1 = jnp.zeros((1,), dtype=jnp.int32)
    one1 = lax.bitcast_convert_type(
        jnp.full((1,), 1.0, jnp.float32), jnp.int32)
    class_vecs = jnp.stack(
        [jnp.concatenate([neg1, sample_ids]),
         jnp.concatenate([zero1,
                          lax.bitcast_convert_type(bias[:ns], jnp.int32)]),
         jnp.concatenate([one1,
                          lax.bitcast_convert_type(sample_freq, jnp.int32)])],
        axis=1)
    logits_t = _tc_logits_t(
        output,
        targets.reshape(1, bsz),
        swp,
        trows,
        class_vecs,
        bias[ns:].reshape(1, bsz),
        true_freq.reshape(1, bsz),
        bt=512,
    )
    logits = logits_t.T
    new_targets = jnp.zeros((bsz,), dtype=jnp.int32)
    return logits, new_targets
